# baseline (speedup 1.0000x reference)
import functools

import jax
import jax.numpy as jnp
from jax import lax
from jax.experimental import pallas as pl
from jax.experimental.pallas import tpu as pltpu
from jax.experimental.pallas import tpu_sc as plsc

N = 10000
E = 320000
NODE_H = 128
GNN_H = 256
HALF = 128

NS = 16
CW = 128
KCH = 160
E2 = NS * KCH * CW
GB = 8
NH = 5120
NHP = NH + 8
NPAD = 2 * NH
RPH = NH // NS


def _p_kernel(x_ref, w_ref, b_ref, p_ref):
    p = (jnp.dot(x_ref[...], w_ref[...], preferred_element_type=jnp.float32)
         + b_ref[...])
    p_ref[0] = p[:, :HALF]
    p_ref[1] = p[:, HALF:]


def _q_kernel(ea_ref, w_ref, q_ref):
    qq = jnp.dot(ea_ref[...], w_ref[...], preferred_element_type=jnp.float32)
    q_ref[0] = qq[:, :HALF]
    q_ref[1] = qq[:, HALF:]


def _sc_body(p_all, q_all, row3d, col3d, acc_out, cnt_out,
             row8, col8, ccl8, gb0, qb0, gb1, qb1, acc_sh, sem0, sem1,
             sem2):
    c = lax.axis_index("c")
    s = lax.axis_index("s")
    bufs = ((gb0, qb0, sem0), (gb1, qb1, sem1))
    gbuf, qbuf = gb0, qb0

    def _fill(buf, val):
        def body(i, _):
            for k in range(HALF // 16):
                buf[i, pl.ds(k * 16, 16)] = jnp.full((16,), val, jnp.float32)
            return 0
        lax.fori_loop(0, CW, body, 0)

    def _zero_acc():
        _fill(qbuf, 0.0)
        pltpu.sync_copy(qbuf, acc_sh.at[pl.ds(s * RPH, CW)])
        pltpu.sync_copy(qbuf, acc_sh.at[pl.ds(s * RPH + CW, CW)])
        pltpu.sync_copy(qbuf.at[pl.ds(0, RPH - 2 * CW)],
                        acc_sh.at[pl.ds(s * RPH + 2 * CW, RPH - 2 * CW)])

    def _transform(base):
        for r in range(GB):
            for k in range(CW // 16):
                sl = pl.ds(k * 16, 16)
                t = col8[r, sl] - base
                ok = (t >= 0) & (t < NH)
                ccl8[r, sl] = jnp.where(ok, t, NH)

    def _pass(h, _):
        base = h * NH
        _zero_acc()
        plsc.subcore_barrier()

        def group(g, _):
            pltpu.sync_copy(row3d.at[s, pl.ds(g * GB, GB)], row8)
            pltpu.sync_copy(col3d.at[s, pl.ds(g * GB, GB)], col8)
            _transform(base)

            def start(jj, b):
                gb, qb, sm = b
                dg = pltpu.async_copy(p_all.at[c].at[row8.at[jj]], gb, sm)
                dq = pltpu.async_copy(
                    q_all.at[c].at[pl.ds((s * KCH + g * GB + jj) * CW, CW)],
                    qb, sm)
                return dg, dq

            def ebody_fn(gb, qb):
                @plsc.parallel_loop(0, CW, unroll=4)
                def _(i):
                    for k in range(HALF // 16):
                        sl = pl.ds(k * 16, 16)
                        gb[i, sl] = jnp.maximum(gb[i, sl] + qb[i, sl], 0.0)

            pend = {0: start(0, bufs[0])}
            scp = {}
            for jj in range(GB):
                if jj + 1 < GB:
                    if jj - 1 in scp:
                        scp.pop(jj - 1).wait()
                    pend[jj + 1] = start(jj + 1, bufs[(jj + 1) % 2])
                dg, dq = pend.pop(jj)
                dg.wait()
                dq.wait()
                gb, qb, _sm = bufs[jj % 2]
                ebody_fn(gb, qb)
                scp[jj] = pltpu.async_copy(gb, acc_sh.at[ccl8.at[jj]],
                                           sem2, add=True)
            scp.pop(GB - 2).wait()
            scp.pop(GB - 1).wait()
            return 0
        lax.fori_loop(0, KCH // GB, group, 0)

        plsc.subcore_barrier()

        pltpu.sync_copy(acc_sh.at[pl.ds(s * RPH, RPH)],
                        acc_out.at[c, pl.ds(base + s * RPH, RPH)])
        return 0

    lax.fori_loop(0, 2, _pass, 0)

    plsc.subcore_barrier()
    base = c * NH
    _zero_acc()
    _fill(gbuf, 1.0)
    plsc.subcore_barrier()

    def cgroup(g, _):
        pltpu.sync_copy(col3d.at[s, pl.ds(g * GB, GB)], col8)
        _transform(base)

        def cchunk(jj, _):
            pltpu.sync_copy(gbuf, acc_sh.at[ccl8.at[jj]], add=True)
            return 0
        lax.fori_loop(0, GB, cchunk, 0)
        return 0
    lax.fori_loop(0, KCH // GB, cgroup, 0)

    plsc.subcore_barrier()
    pltpu.sync_copy(acc_sh.at[pl.ds(s * RPH, RPH)],
                    cnt_out.at[pl.ds(base + s * RPH, RPH)])


_sc_scatter = functools.partial(
    pl.kernel,
    out_type=(
        jax.ShapeDtypeStruct((2, NPAD, HALF), jnp.float32),
        jax.ShapeDtypeStruct((NPAD, HALF), jnp.float32),
    ),
    mesh=plsc.VectorSubcoreMesh(core_axis_name="c", subcore_axis_name="s"),
    scratch_types=[
        pltpu.VMEM((GB, CW), jnp.int32),
        pltpu.VMEM((GB, CW), jnp.int32),
        pltpu.VMEM((GB, CW), jnp.int32),
        pltpu.VMEM((CW, HALF), jnp.float32),
        pltpu.VMEM((CW, HALF), jnp.float32),
        pltpu.VMEM((CW, HALF), jnp.float32),
        pltpu.VMEM((CW, HALF), jnp.float32),
        pltpu.VMEM_SHARED((NHP, HALF), jnp.float32),
        pltpu.SemaphoreType.DMA,
        pltpu.SemaphoreType.DMA,
        pltpu.SemaphoreType.DMA,
    ],
)(_sc_body)


def _final_kernel(x_ref, acc_ref, cnt_ref, wa_ref, wb0_ref, wb1_ref, b_ref,
                  out_ref):
    inv = 1.0 / jnp.maximum(cnt_ref[...], 1.0)[:, 0:1]
    m0 = acc_ref[0] * inv
    m1 = acc_ref[1] * inv
    h = (jnp.dot(x_ref[...], wa_ref[...], preferred_element_type=jnp.float32)
         + jnp.dot(m0, wb0_ref[...], preferred_element_type=jnp.float32)
         + jnp.dot(m1, wb1_ref[...], preferred_element_type=jnp.float32)
         + b_ref[...])
    out_ref[...] = jnp.maximum(h, 0.0)


def kernel(x, edge_index, edge_attr, W1, b1, W2, b2):
    pad = E2 - E
    row_pad = jnp.concatenate(
        [edge_index[0], jnp.zeros((pad,), jnp.int32)]).reshape(NS, KCH, CW)
    col_pad = jnp.concatenate(
        [edge_index[1], jnp.full((pad,), 2 * NH, jnp.int32)]
    ).reshape(NS, KCH, CW)
    ea_pad = jnp.concatenate(
        [edge_attr, jnp.zeros((pad, edge_attr.shape[1]), edge_attr.dtype)])

    b1r = b1.reshape(1, GNN_H)
    b2r = b2.reshape(1, GNN_H)

    p_all = pl.pallas_call(
        _p_kernel,
        out_shape=jax.ShapeDtypeStruct((2, N, HALF), jnp.float32),
    )(x, W1[:NODE_H], b1r)

    BE = 8192
    q_all = pl.pallas_call(
        _q_kernel,
        grid=(E2 // BE,),
        in_specs=[
            pl.BlockSpec((BE, 16), lambda i: (i, 0)),
            pl.BlockSpec((16, GNN_H), lambda i: (0, 0)),
        ],
        out_specs=pl.BlockSpec((2, BE, HALF), lambda i: (0, i, 0)),
        out_shape=jax.ShapeDtypeStruct((2, E2, HALF), jnp.float32),
    )(ea_pad, W1[NODE_H:])

    acc, cnt = _sc_scatter(p_all, q_all, row_pad, col_pad)

    BN = 2000
    out = pl.pallas_call(
        _final_kernel,
        grid=(N // BN,),
        in_specs=[
            pl.BlockSpec((BN, NODE_H), lambda i: (i, 0)),
            pl.BlockSpec((2, BN, HALF), lambda i: (0, i, 0)),
            pl.BlockSpec((BN, HALF), lambda i: (i, 0)),
            pl.BlockSpec((NODE_H, GNN_H), lambda i: (0, 0)),
            pl.BlockSpec((HALF, GNN_H), lambda i: (0, 0)),
            pl.BlockSpec((HALF, GNN_H), lambda i: (0, 0)),
            pl.BlockSpec((1, GNN_H), lambda i: (0, 0)),
        ],
        out_specs=pl.BlockSpec((BN, GNN_H), lambda i: (i, 0)),
        out_shape=jax.ShapeDtypeStruct((N, GNN_H), jnp.float32),
    )(x, acc, cnt, W2[:NODE_H], W2[NODE_H:NODE_H + HALF],
      W2[NODE_H + HALF:], b2r)

    return out


# speedup vs baseline: 1.0469x; 1.0469x over previous
import functools

import jax
import jax.numpy as jnp
from jax import lax
from jax.experimental import pallas as pl
from jax.experimental.pallas import tpu as pltpu
from jax.experimental.pallas import tpu_sc as plsc

N = 10000
E = 320000
NODE_H = 128
GNN_H = 256
HALF = 128

NS = 16
CW = 128
KCH = 160
E2 = NS * KCH * CW
GB = 8
NH = 5120
NHP = NH + 8
NPAD = 2 * NH
RPH = NH // NS


def _p_kernel(x_ref, w_ref, b_ref, p_ref):
    p = (jnp.dot(x_ref[...], w_ref[...], preferred_element_type=jnp.float32)
         + b_ref[...])
    p_ref[0] = p[:, :HALF]
    p_ref[1] = p[:, HALF:]


def _q_kernel(ea_ref, w_ref, q_ref):
    qq = jnp.dot(ea_ref[...], w_ref[...], preferred_element_type=jnp.float32)
    q_ref[0] = qq[:, :HALF]
    q_ref[1] = qq[:, HALF:]


def _sc_body(p_all, q_all, row3d, col3d, acc_out, cnt_out,
             row8, col8, ccl8, gb0, gb1, gb2, qb0, qb1, acc_sh,
             sg0, sg1, sg2, sq0, sq1, ss0, ss1, ss2):
    c = lax.axis_index("c")
    s = lax.axis_index("s")
    gbufs = (gb0, gb1, gb2)
    qbufs = (qb0, qb1)
    gsems = (sg0, sg1, sg2)
    qsems = (sq0, sq1)
    ssems = (ss0, ss1, ss2)
    gbuf, qbuf = gb0, qb0

    def _fill(buf, val):
        def body(i, _):
            for k in range(HALF // 16):
                buf[i, pl.ds(k * 16, 16)] = jnp.full((16,), val, jnp.float32)
            return 0
        lax.fori_loop(0, CW, body, 0)

    def _zero_acc():
        _fill(qbuf, 0.0)
        pltpu.sync_copy(qbuf, acc_sh.at[pl.ds(s * RPH, CW)])
        pltpu.sync_copy(qbuf, acc_sh.at[pl.ds(s * RPH + CW, CW)])
        pltpu.sync_copy(qbuf.at[pl.ds(0, RPH - 2 * CW)],
                        acc_sh.at[pl.ds(s * RPH + 2 * CW, RPH - 2 * CW)])

    def _transform(base):
        for r in range(GB):
            for k in range(CW // 16):
                sl = pl.ds(k * 16, 16)
                t = col8[r, sl] - base
                ok = (t >= 0) & (t < NH)
                ccl8[r, sl] = jnp.where(ok, t, NH)

    def _pass(h, _):
        base = h * NH
        _zero_acc()
        plsc.subcore_barrier()

        def group(g, _):
            pltpu.sync_copy(row3d.at[s, pl.ds(g * GB, GB)], row8)
            pltpu.sync_copy(col3d.at[s, pl.ds(g * GB, GB)], col8)
            _transform(base)

            def start_g(jj):
                b = jj % 3
                return pltpu.async_copy(p_all.at[c].at[row8.at[jj]],
                                        gbufs[b], gsems[b])

            def start_q(jj):
                b = jj % 2
                return pltpu.async_copy(
                    q_all.at[c].at[pl.ds((s * KCH + g * GB + jj) * CW, CW)],
                    qbufs[b], qsems[b])

            def ebody_fn(gb, qb):
                @plsc.parallel_loop(0, CW, unroll=4)
                def _(i):
                    for k in range(HALF // 16):
                        sl = pl.ds(k * 16, 16)
                        gb[i, sl] = jnp.maximum(gb[i, sl] + qb[i, sl], 0.0)

            gp = {0: start_g(0), 1: start_g(1)}
            qp = {0: start_q(0)}
            scp = {}
            for jj in range(GB):
                if jj + 1 < GB:
                    qp[jj + 1] = start_q(jj + 1)
                if jj + 2 < GB:
                    if jj - 1 in scp:
                        scp.pop(jj - 1).wait()
                    gp[jj + 2] = start_g(jj + 2)
                gp.pop(jj).wait()
                qp.pop(jj).wait()
                gb = gbufs[jj % 3]
                ebody_fn(gb, qbufs[jj % 2])
                scp[jj] = pltpu.async_copy(gb, acc_sh.at[ccl8.at[jj]],
                                           ssems[jj % 3], add=True)
            for r in sorted(scp):
                scp.pop(r).wait()
            return 0
        lax.fori_loop(0, KCH // GB, group, 0)

        plsc.subcore_barrier()

        pltpu.sync_copy(acc_sh.at[pl.ds(s * RPH, RPH)],
                        acc_out.at[c, pl.ds(base + s * RPH, RPH)])
        return 0

    lax.fori_loop(0, 2, _pass, 0)

    plsc.subcore_barrier()
    base = c * NH
    _zero_acc()
    _fill(gbuf, 1.0)
    plsc.subcore_barrier()

    def cgroup(g, _):
        pltpu.sync_copy(col3d.at[s, pl.ds(g * GB, GB)], col8)
        _transform(base)

        def cchunk(jj, _):
            pltpu.sync_copy(gbuf, acc_sh.at[ccl8.at[jj]], add=True)
            return 0
        lax.fori_loop(0, GB, cchunk, 0)
        return 0
    lax.fori_loop(0, KCH // GB, cgroup, 0)

    plsc.subcore_barrier()
    pltpu.sync_copy(acc_sh.at[pl.ds(s * RPH, RPH)],
                    cnt_out.at[pl.ds(base + s * RPH, RPH)])


_sc_scatter = functools.partial(
    pl.kernel,
    out_type=(
        jax.ShapeDtypeStruct((2, NPAD, HALF), jnp.float32),
        jax.ShapeDtypeStruct((NPAD, HALF), jnp.float32),
    ),
    mesh=plsc.VectorSubcoreMesh(core_axis_name="c", subcore_axis_name="s"),
    scratch_types=[
        pltpu.VMEM((GB, CW), jnp.int32),
        pltpu.VMEM((GB, CW), jnp.int32),
        pltpu.VMEM((GB, CW), jnp.int32),
        pltpu.VMEM((CW, HALF), jnp.float32),
        pltpu.VMEM((CW, HALF), jnp.float32),
        pltpu.VMEM((CW, HALF), jnp.float32),
        pltpu.VMEM((CW, HALF), jnp.float32),
        pltpu.VMEM((CW, HALF), jnp.float32),
        pltpu.VMEM_SHARED((NHP, HALF), jnp.float32),
    ] + [pltpu.SemaphoreType.DMA] * 8,
)(_sc_body)


def _final_kernel(x_ref, acc_ref, cnt_ref, wa_ref, wb0_ref, wb1_ref, b_ref,
                  out_ref):
    inv = 1.0 / jnp.maximum(cnt_ref[...], 1.0)[:, 0:1]
    m0 = acc_ref[0] * inv
    m1 = acc_ref[1] * inv
    h = (jnp.dot(x_ref[...], wa_ref[...], preferred_element_type=jnp.float32)
         + jnp.dot(m0, wb0_ref[...], preferred_element_type=jnp.float32)
         + jnp.dot(m1, wb1_ref[...], preferred_element_type=jnp.float32)
         + b_ref[...])
    out_ref[...] = jnp.maximum(h, 0.0)


def kernel(x, edge_index, edge_attr, W1, b1, W2, b2):
    pad = E2 - E
    row_pad = jnp.concatenate(
        [edge_index[0], jnp.zeros((pad,), jnp.int32)]).reshape(NS, KCH, CW)
    col_pad = jnp.concatenate(
        [edge_index[1], jnp.full((pad,), 2 * NH, jnp.int32)]
    ).reshape(NS, KCH, CW)
    ea_pad = jnp.concatenate(
        [edge_attr, jnp.zeros((pad, edge_attr.shape[1]), edge_attr.dtype)])

    b1r = b1.reshape(1, GNN_H)
    b2r = b2.reshape(1, GNN_H)

    p_all = pl.pallas_call(
        _p_kernel,
        out_shape=jax.ShapeDtypeStruct((2, N, HALF), jnp.float32),
    )(x, W1[:NODE_H], b1r)

    BE = 8192
    q_all = pl.pallas_call(
        _q_kernel,
        grid=(E2 // BE,),
        in_specs=[
            pl.BlockSpec((BE, 16), lambda i: (i, 0)),
            pl.BlockSpec((16, GNN_H), lambda i: (0, 0)),
        ],
        out_specs=pl.BlockSpec((2, BE, HALF), lambda i: (0, i, 0)),
        out_shape=jax.ShapeDtypeStruct((2, E2, HALF), jnp.float32),
    )(ea_pad, W1[NODE_H:])

    acc, cnt = _sc_scatter(p_all, q_all, row_pad, col_pad)

    BN = 2000
    out = pl.pallas_call(
        _final_kernel,
        grid=(N // BN,),
        in_specs=[
            pl.BlockSpec((BN, NODE_H), lambda i: (i, 0)),
            pl.BlockSpec((2, BN, HALF), lambda i: (0, i, 0)),
            pl.BlockSpec((BN, HALF), lambda i: (i, 0)),
            pl.BlockSpec((NODE_H, GNN_H), lambda i: (0, 0)),
            pl.BlockSpec((HALF, GNN_H), lambda i: (0, 0)),
            pl.BlockSpec((HALF, GNN_H), lambda i: (0, 0)),
            pl.BlockSpec((1, GNN_H), lambda i: (0, 0)),
        ],
        out_specs=pl.BlockSpec((BN, GNN_H), lambda i: (i, 0)),
        out_shape=jax.ShapeDtypeStruct((N, GNN_H), jnp.float32),
    )(x, acc, cnt, W2[:NODE_H], W2[NODE_H:NODE_H + HALF],
      W2[NODE_H + HALF:], b2r)

    return out
